# baseline (speedup 1.0000x reference)
import functools

import jax
import jax.numpy as jnp
from jax import lax
from jax.experimental import pallas as pl
from jax.experimental.pallas import tpu as pltpu
from jax.experimental.pallas import tpu_sc as plsc

N = 10000
D = 128
E = 320000

NC = 2
NS = 16
NW = NC * NS
EPW = E // NW
RSTRIDE = 624
RSPAN = 640
C = 128
NCHUNK = EPW // C
TAIL = EPW - NCHUNK * C


def _scat_body(
    g_hbm, src_hbm, dst_hbm, out_hbm,
    si0, si1, si2, si3, di0, di1, di2, di3, st, dt,
    rows0, rows1, rowst,
    acc, is0, is1, is2, is3, gs0, gs1, ss0, ss1,
):
    sidx = (si0, si1, si2, si3)
    didx = (di0, di1, di2, di3)
    rows = (rows0, rows1)
    isem = (is0, is1, is2, is3)
    gsem = (gs0, gs1)
    ssem = (ss0, ss1)
    c = lax.axis_index("c")
    s = lax.axis_index("s")
    base = (c * NS + s) * EPW

    def ioff(ch):
        return pl.multiple_of(base + ch * C, 8)

    def idx_sync(ch, q):
        pltpu.sync_copy(src_hbm.at[pl.ds(ioff(ch), C)], sidx[q])
        pltpu.sync_copy(dst_hbm.at[pl.ds(ioff(ch), C)], didx[q])

    def idx_start(ch, q):
        pltpu.async_copy(src_hbm.at[pl.ds(ioff(ch), C)], sidx[q], isem[q])
        pltpu.async_copy(dst_hbm.at[pl.ds(ioff(ch), C)], didx[q], isem[q])

    def idx_wait(ch, q):
        pltpu.make_async_copy(src_hbm.at[pl.ds(ioff(ch), C)], sidx[q], isem[q]).wait()
        pltpu.make_async_copy(dst_hbm.at[pl.ds(ioff(ch), C)], didx[q], isem[q]).wait()

    def gather_start(q, r):
        pltpu.async_copy(g_hbm.at[sidx[q]], rows[r], gsem[r])

    def gather_wait(q, r):
        pltpu.make_async_copy(g_hbm.at[sidx[q]], rows[r], gsem[r]).wait()

    def scat_start(q, r):
        pltpu.async_copy(rows[r], acc.at[didx[q]], ssem[r], add=True)

    def scat_wait(q, r):
        pltpu.make_async_copy(rows[r], acc.at[didx[q]], ssem[r]).wait()

    idx_sync(0, 0)
    idx_sync(1, 1)
    idx_start(2, 2)
    idx_start(3, 3)
    gather_start(0, 0)
    r0 = pl.multiple_of(s * RSTRIDE, 8)
    pltpu.sync_copy(g_hbm.at[pl.ds(r0, RSPAN)], acc.at[pl.ds(r0, RSPAN)])
    plsc.subcore_barrier()
    gather_start(1, 1)
    gather_wait(0, 0)
    scat_start(0, 0)

    def body(g, carry):
        for b in range(4):
            ch = 4 * g + 2 + b
            rcur = b % 2
            qcur = (2 + b) % 4
            scat_wait(b % 4, rcur)
            idx_wait(ch, qcur)
            gather_start(qcur, rcur)
            gather_wait((1 + b) % 4, 1 - rcur)
            scat_start((1 + b) % 4, 1 - rcur)
            idx_start(ch + 2, b % 4)
        return carry

    lax.fori_loop(0, (NCHUNK - 6) // 4, body, 0)

    for k in range(4):
        ch = NCHUNK - 4 + k
        rcur = ch % 2
        qcur = ch % 4
        scat_wait((ch - 2) % 4, rcur)
        idx_wait(ch, qcur)
        gather_start(qcur, rcur)
        gather_wait((ch - 1) % 4, 1 - rcur)
        scat_start((ch - 1) % 4, 1 - rcur)
        if k < 2:
            idx_start(ch + 2, (ch + 2) % 4)
    gather_wait((NCHUNK - 1) % 4, (NCHUNK - 1) % 2)
    scat_start((NCHUNK - 1) % 4, (NCHUNK - 1) % 2)
    scat_wait((NCHUNK - 2) % 4, (NCHUNK - 2) % 2)
    scat_wait((NCHUNK - 1) % 4, (NCHUNK - 1) % 2)
    offt = pl.multiple_of(base + NCHUNK * C, 8)
    pltpu.sync_copy(src_hbm.at[pl.ds(offt, TAIL)], st)
    pltpu.sync_copy(dst_hbm.at[pl.ds(offt, TAIL)], dt)
    pltpu.async_copy(g_hbm.at[st], rowst, gs0).wait()
    pltpu.sync_copy(rowst, acc.at[dt], add=True)
    plsc.subcore_barrier()
    pltpu.sync_copy(
        acc.at[pl.ds(r0, RSPAN)], out_hbm.at[c, pl.ds(r0, RSPAN)]
    )


@functools.cache
def _scat_kernel():
    mesh = plsc.VectorSubcoreMesh(
        core_axis_name="c", subcore_axis_name="s", num_cores=NC, num_subcores=NS
    )
    return pl.kernel(
        _scat_body,
        out_type=jax.ShapeDtypeStruct((NC, N, D), jnp.float32),
        mesh=mesh,
        scratch_types=[
            pltpu.VMEM((C,), jnp.int32),
            pltpu.VMEM((C,), jnp.int32),
            pltpu.VMEM((C,), jnp.int32),
            pltpu.VMEM((C,), jnp.int32),
            pltpu.VMEM((C,), jnp.int32),
            pltpu.VMEM((C,), jnp.int32),
            pltpu.VMEM((C,), jnp.int32),
            pltpu.VMEM((C,), jnp.int32),
            pltpu.VMEM((TAIL,), jnp.int32),
            pltpu.VMEM((TAIL,), jnp.int32),
            pltpu.VMEM((C, D), jnp.float32),
            pltpu.VMEM((C, D), jnp.float32),
            pltpu.VMEM((TAIL, D), jnp.float32),
            pltpu.VMEM_SHARED((N, D), jnp.float32),
            pltpu.SemaphoreType.DMA,
            pltpu.SemaphoreType.DMA,
            pltpu.SemaphoreType.DMA,
            pltpu.SemaphoreType.DMA,
            pltpu.SemaphoreType.DMA,
            pltpu.SemaphoreType.DMA,
            pltpu.SemaphoreType.DMA,
            pltpu.SemaphoreType.DMA,
        ],
    )


BR = 400
NB = N // BR


def _pre_body(x_ref, w_ref, deg_ref, g_ref, dinv_ref):
    deg = deg_ref[0, :, 0:1] + deg_ref[1, :, 0:1] - 1.0
    dinv = lax.rsqrt(deg)
    g_ref[...] = (
        jnp.dot(x_ref[...], w_ref[...], preferred_element_type=jnp.float32) * dinv
    )
    dinv_ref[...] = jnp.broadcast_to(dinv, (BR, 16))


_pre_kernel = pl.pallas_call(
    _pre_body,
    grid=(NB,),
    in_specs=[
        pl.BlockSpec((BR, D), lambda i: (i, 0)),
        pl.BlockSpec((D, D), lambda i: (0, 0)),
        pl.BlockSpec((NC, BR, D), lambda i: (0, i, 0)),
    ],
    out_specs=[
        pl.BlockSpec((BR, D), lambda i: (i, 0)),
        pl.BlockSpec((BR, 16), lambda i: (i, 0)),
    ],
    out_shape=[
        jax.ShapeDtypeStruct((N, D), jnp.float32),
        jax.ShapeDtypeStruct((N, 16), jnp.float32),
    ],
)


def _mid_body(part_ref, g_ref, dinv_ref, b_ref, w_ref, gn_ref):
    dinv = dinv_ref[:, 0:1]
    agg = part_ref[0] + part_ref[1] - g_ref[...]
    xn = jnp.maximum(agg * dinv + b_ref[...], 0.0)
    gn_ref[...] = (
        jnp.dot(xn, w_ref[...], preferred_element_type=jnp.float32) * dinv
    )


_mid_kernel = pl.pallas_call(
    _mid_body,
    grid=(NB,),
    in_specs=[
        pl.BlockSpec((NC, BR, D), lambda i: (0, i, 0)),
        pl.BlockSpec((BR, D), lambda i: (i, 0)),
        pl.BlockSpec((BR, 16), lambda i: (i, 0)),
        pl.BlockSpec((1, D), lambda i: (0, 0)),
        pl.BlockSpec((D, D), lambda i: (0, 0)),
    ],
    out_specs=pl.BlockSpec((BR, D), lambda i: (i, 0)),
    out_shape=jax.ShapeDtypeStruct((N, D), jnp.float32),
)


def _fin_body(part_ref, g_ref, dinv_ref, b_ref, x0_ref, wl_ref, bl_ref,
              h_ref, out_ref):
    i = pl.program_id(0)
    dinv = dinv_ref[:, 0:1]
    agg = part_ref[0] + part_ref[1] - g_ref[...]
    h = jnp.maximum(agg * dinv + b_ref[...], 0.0) + x0_ref[...]
    h_ref[...] = h

    @pl.when(i == 0)
    def _():
        out_ref[...] = jnp.zeros_like(out_ref)

    out_ref[...] += jnp.sum(h, axis=0, keepdims=True)

    @pl.when(i == NB - 1)
    def _():
        out_ref[...] = (
            jnp.dot(out_ref[...] * (1.0 / N), wl_ref[...],
                    preferred_element_type=jnp.float32)
            + bl_ref[...]
        )


_fin_kernel = pl.pallas_call(
    _fin_body,
    grid=(NB,),
    in_specs=[
        pl.BlockSpec((NC, BR, D), lambda i: (0, i, 0)),
        pl.BlockSpec((BR, D), lambda i: (i, 0)),
        pl.BlockSpec((BR, 16), lambda i: (i, 0)),
        pl.BlockSpec((1, D), lambda i: (0, 0)),
        pl.BlockSpec((BR, D), lambda i: (i, 0)),
        pl.BlockSpec((D, D), lambda i: (0, 0)),
        pl.BlockSpec((1, D), lambda i: (0, 0)),
    ],
    out_specs=[
        pl.BlockSpec((BR, D), lambda i: (i, 0)),
        pl.BlockSpec((1, D), lambda i: (0, 0)),
    ],
    out_shape=[
        jax.ShapeDtypeStruct((N, D), jnp.float32),
        jax.ShapeDtypeStruct((1, D), jnp.float32),
    ],
)


def kernel(graph_x, graph_edge, W1, b1, W2, b2, W3, b3, Wl, bl):
    edge = graph_edge.astype(jnp.int32)
    src = edge[0]
    dst = edge[1]

    deg2 = _scat_kernel()(jnp.ones((N, D), jnp.float32), dst, dst)
    g1, dinv16 = _pre_kernel(graph_x, W1, deg2)
    p1 = _scat_kernel()(g1, src, dst)
    g2 = _mid_kernel(p1, g1, dinv16, b1.reshape(1, D), W2)
    p2 = _scat_kernel()(g2, src, dst)
    g3 = _mid_kernel(p2, g2, dinv16, b2.reshape(1, D), W3)
    p3 = _scat_kernel()(g3, src, dst)
    h, out = _fin_kernel(
        p3, g3, dinv16, b3.reshape(1, D), graph_x, Wl, bl.reshape(1, D)
    )
    return (h, out)


# speedup vs baseline: 1.0564x; 1.0564x over previous
import functools

import jax
import jax.numpy as jnp
from jax import lax
from jax.experimental import pallas as pl
from jax.experimental.pallas import tpu as pltpu
from jax.experimental.pallas import tpu_sc as plsc

N = 10000
D = 128
E = 320000

NC = 2
NS = 16
NW = NC * NS
EPW = E // NW
RSTRIDE = 624
RSPAN = 640
C = 128
NCHUNK = EPW // C
TAIL = EPW - NCHUNK * C


def _deg_body(ones_hbm, dst_hbm, out_hbm, di0, di1, di2, di3, dt,
              ones_v, acc, is0, is1, is2, is3, ss0, ss1):
    didx = (di0, di1, di2, di3)
    isem = (is0, is1, is2, is3)
    ssem = (ss0, ss1)
    c = lax.axis_index("c")
    s = lax.axis_index("s")
    base = (c * NS + s) * EPW
    r0 = pl.multiple_of(s * RSTRIDE, 8)
    pltpu.sync_copy(ones_hbm.at[pl.ds(0, C)], ones_v)
    pltpu.sync_copy(ones_hbm.at[pl.ds(r0, RSPAN)], acc.at[pl.ds(r0, RSPAN)])
    plsc.subcore_barrier()

    def ioff(ch):
        return pl.multiple_of(base + ch * C, 8)

    def idx_sync(ch, q):
        pltpu.sync_copy(dst_hbm.at[pl.ds(ioff(ch), C)], didx[q])

    def idx_start(ch, q):
        pltpu.async_copy(dst_hbm.at[pl.ds(ioff(ch), C)], didx[q], isem[q])

    def idx_wait(ch, q):
        pltpu.make_async_copy(dst_hbm.at[pl.ds(ioff(ch), C)], didx[q], isem[q]).wait()

    def scat_start(q, r):
        pltpu.async_copy(ones_v, acc.at[didx[q]], ssem[r], add=True)

    def scat_wait(q, r):
        pltpu.make_async_copy(ones_v, acc.at[didx[q]], ssem[r]).wait()

    idx_sync(0, 0)
    idx_sync(1, 1)
    idx_start(2, 2)
    idx_start(3, 3)
    scat_start(0, 0)
    scat_start(1, 1)

    def body(g, carry):
        for b in range(4):
            ch = 4 * g + 2 + b
            scat_wait(b % 4, b % 2)
            idx_wait(ch, (2 + b) % 4)
            scat_start((2 + b) % 4, b % 2)
            idx_start(ch + 2, b % 4)
        return carry

    lax.fori_loop(0, (NCHUNK - 6) // 4, body, 0)
    for k in range(4):
        ch = NCHUNK - 4 + k
        scat_wait((ch - 2) % 4, ch % 2)
        idx_wait(ch, ch % 4)
        scat_start(ch % 4, ch % 2)
        if k < 2:
            idx_start(ch + 2, (ch + 2) % 4)
    scat_wait((NCHUNK - 2) % 4, (NCHUNK - 2) % 2)
    scat_wait((NCHUNK - 1) % 4, (NCHUNK - 1) % 2)
    offt = pl.multiple_of(base + NCHUNK * C, 8)
    pltpu.sync_copy(dst_hbm.at[pl.ds(offt, TAIL)], dt)
    pltpu.sync_copy(ones_v.at[pl.ds(0, TAIL)], acc.at[dt], add=True)
    plsc.subcore_barrier()
    pltpu.sync_copy(
        acc.at[pl.ds(r0, RSPAN)], out_hbm.at[c, pl.ds(r0, RSPAN)]
    )


@functools.cache
def _deg_kernel():
    mesh = plsc.VectorSubcoreMesh(
        core_axis_name="c", subcore_axis_name="s", num_cores=NC, num_subcores=NS
    )
    return pl.kernel(
        _deg_body,
        out_type=jax.ShapeDtypeStruct((NC, N, D), jnp.float32),
        mesh=mesh,
        scratch_types=[
            pltpu.VMEM((C,), jnp.int32),
            pltpu.VMEM((C,), jnp.int32),
            pltpu.VMEM((C,), jnp.int32),
            pltpu.VMEM((C,), jnp.int32),
            pltpu.VMEM((TAIL,), jnp.int32),
            pltpu.VMEM((C, D), jnp.float32),
            pltpu.VMEM_SHARED((N, D), jnp.float32),
            pltpu.SemaphoreType.DMA,
            pltpu.SemaphoreType.DMA,
            pltpu.SemaphoreType.DMA,
            pltpu.SemaphoreType.DMA,
            pltpu.SemaphoreType.DMA,
            pltpu.SemaphoreType.DMA,
        ],
    )


def _deg_from_raw(raw):
    return (raw[0, :, 0] + raw[1, :, 0] - 2.0).reshape(N, 1)


def _scat_body(
    g_hbm, src_hbm, dst_hbm, out_hbm,
    si0, si1, si2, si3, di0, di1, di2, di3, st, dt,
    rows0, rows1, rowst,
    acc, is0, is1, is2, is3, gs0, gs1, ss0, ss1,
):
    sidx = (si0, si1, si2, si3)
    didx = (di0, di1, di2, di3)
    rows = (rows0, rows1)
    isem = (is0, is1, is2, is3)
    gsem = (gs0, gs1)
    ssem = (ss0, ss1)
    c = lax.axis_index("c")
    s = lax.axis_index("s")
    base = (c * NS + s) * EPW

    def ioff(ch):
        return pl.multiple_of(base + ch * C, 8)

    def idx_sync(ch, q):
        pltpu.sync_copy(src_hbm.at[pl.ds(ioff(ch), C)], sidx[q])
        pltpu.sync_copy(dst_hbm.at[pl.ds(ioff(ch), C)], didx[q])

    def idx_start(ch, q):
        pltpu.async_copy(src_hbm.at[pl.ds(ioff(ch), C)], sidx[q], isem[q])
        pltpu.async_copy(dst_hbm.at[pl.ds(ioff(ch), C)], didx[q], isem[q])

    def idx_wait(ch, q):
        pltpu.make_async_copy(src_hbm.at[pl.ds(ioff(ch), C)], sidx[q], isem[q]).wait()
        pltpu.make_async_copy(dst_hbm.at[pl.ds(ioff(ch), C)], didx[q], isem[q]).wait()

    def gather_start(q, r):
        pltpu.async_copy(g_hbm.at[sidx[q]], rows[r], gsem[r])

    def gather_wait(q, r):
        pltpu.make_async_copy(g_hbm.at[sidx[q]], rows[r], gsem[r]).wait()

    def scat_start(q, r):
        pltpu.async_copy(rows[r], acc.at[didx[q]], ssem[r], add=True)

    def scat_wait(q, r):
        pltpu.make_async_copy(rows[r], acc.at[didx[q]], ssem[r]).wait()

    idx_sync(0, 0)
    idx_sync(1, 1)
    idx_start(2, 2)
    idx_start(3, 3)
    gather_start(0, 0)
    r0 = pl.multiple_of(s * RSTRIDE, 8)
    pltpu.sync_copy(g_hbm.at[pl.ds(r0, RSPAN)], acc.at[pl.ds(r0, RSPAN)])
    plsc.subcore_barrier()
    gather_start(1, 1)
    gather_wait(0, 0)
    scat_start(0, 0)

    def body(g, carry):
        for b in range(4):
            ch = 4 * g + 2 + b
            rcur = b % 2
            qcur = (2 + b) % 4
            scat_wait(b % 4, rcur)
            idx_wait(ch, qcur)
            gather_start(qcur, rcur)
            gather_wait((1 + b) % 4, 1 - rcur)
            scat_start((1 + b) % 4, 1 - rcur)
            idx_start(ch + 2, b % 4)
        return carry

    lax.fori_loop(0, (NCHUNK - 6) // 4, body, 0)

    for k in range(4):
        ch = NCHUNK - 4 + k
        rcur = ch % 2
        qcur = ch % 4
        scat_wait((ch - 2) % 4, rcur)
        idx_wait(ch, qcur)
        gather_start(qcur, rcur)
        gather_wait((ch - 1) % 4, 1 - rcur)
        scat_start((ch - 1) % 4, 1 - rcur)
        if k < 2:
            idx_start(ch + 2, (ch + 2) % 4)
    gather_wait((NCHUNK - 1) % 4, (NCHUNK - 1) % 2)
    scat_start((NCHUNK - 1) % 4, (NCHUNK - 1) % 2)
    scat_wait((NCHUNK - 2) % 4, (NCHUNK - 2) % 2)
    scat_wait((NCHUNK - 1) % 4, (NCHUNK - 1) % 2)
    offt = pl.multiple_of(base + NCHUNK * C, 8)
    pltpu.sync_copy(src_hbm.at[pl.ds(offt, TAIL)], st)
    pltpu.sync_copy(dst_hbm.at[pl.ds(offt, TAIL)], dt)
    pltpu.async_copy(g_hbm.at[st], rowst, gs0).wait()
    pltpu.sync_copy(rowst, acc.at[dt], add=True)
    plsc.subcore_barrier()
    pltpu.sync_copy(
        acc.at[pl.ds(r0, RSPAN)], out_hbm.at[c, pl.ds(r0, RSPAN)]
    )


@functools.cache
def _scat_kernel():
    mesh = plsc.VectorSubcoreMesh(
        core_axis_name="c", subcore_axis_name="s", num_cores=NC, num_subcores=NS
    )
    return pl.kernel(
        _scat_body,
        out_type=jax.ShapeDtypeStruct((NC, N, D), jnp.float32),
        mesh=mesh,
        scratch_types=[
            pltpu.VMEM((C,), jnp.int32),
            pltpu.VMEM((C,), jnp.int32),
            pltpu.VMEM((C,), jnp.int32),
            pltpu.VMEM((C,), jnp.int32),
            pltpu.VMEM((C,), jnp.int32),
            pltpu.VMEM((C,), jnp.int32),
            pltpu.VMEM((C,), jnp.int32),
            pltpu.VMEM((C,), jnp.int32),
            pltpu.VMEM((TAIL,), jnp.int32),
            pltpu.VMEM((TAIL,), jnp.int32),
            pltpu.VMEM((C, D), jnp.float32),
            pltpu.VMEM((C, D), jnp.float32),
            pltpu.VMEM((TAIL, D), jnp.float32),
            pltpu.VMEM_SHARED((N, D), jnp.float32),
            pltpu.SemaphoreType.DMA,
            pltpu.SemaphoreType.DMA,
            pltpu.SemaphoreType.DMA,
            pltpu.SemaphoreType.DMA,
            pltpu.SemaphoreType.DMA,
            pltpu.SemaphoreType.DMA,
            pltpu.SemaphoreType.DMA,
            pltpu.SemaphoreType.DMA,
        ],
    )


BR = 400
NB = N // BR


def _pre_body(x_ref, w_ref, deg_ref, g_ref, dinv_ref):
    dinv = lax.rsqrt(deg_ref[...] + 1.0)
    g_ref[...] = (
        jnp.dot(x_ref[...], w_ref[...], preferred_element_type=jnp.float32) * dinv
    )
    dinv_ref[...] = jnp.broadcast_to(dinv, (BR, 16))


_pre_kernel = pl.pallas_call(
    _pre_body,
    grid=(NB,),
    in_specs=[
        pl.BlockSpec((BR, D), lambda i: (i, 0)),
        pl.BlockSpec((D, D), lambda i: (0, 0)),
        pl.BlockSpec((BR, 1), lambda i: (i, 0)),
    ],
    out_specs=[
        pl.BlockSpec((BR, D), lambda i: (i, 0)),
        pl.BlockSpec((BR, 16), lambda i: (i, 0)),
    ],
    out_shape=[
        jax.ShapeDtypeStruct((N, D), jnp.float32),
        jax.ShapeDtypeStruct((N, 16), jnp.float32),
    ],
)


def _mid_body(part_ref, g_ref, dinv_ref, b_ref, w_ref, gn_ref):
    dinv = dinv_ref[:, 0:1]
    agg = part_ref[0] + part_ref[1] - g_ref[...]
    xn = jnp.maximum(agg * dinv + b_ref[...], 0.0)
    gn_ref[...] = (
        jnp.dot(xn, w_ref[...], preferred_element_type=jnp.float32) * dinv
    )


_mid_kernel = pl.pallas_call(
    _mid_body,
    grid=(NB,),
    in_specs=[
        pl.BlockSpec((NC, BR, D), lambda i: (0, i, 0)),
        pl.BlockSpec((BR, D), lambda i: (i, 0)),
        pl.BlockSpec((BR, 16), lambda i: (i, 0)),
        pl.BlockSpec((1, D), lambda i: (0, 0)),
        pl.BlockSpec((D, D), lambda i: (0, 0)),
    ],
    out_specs=pl.BlockSpec((BR, D), lambda i: (i, 0)),
    out_shape=jax.ShapeDtypeStruct((N, D), jnp.float32),
)


def _fin_body(part_ref, g_ref, dinv_ref, b_ref, x0_ref, wl_ref, bl_ref,
              h_ref, out_ref):
    i = pl.program_id(0)
    dinv = dinv_ref[:, 0:1]
    agg = part_ref[0] + part_ref[1] - g_ref[...]
    h = jnp.maximum(agg * dinv + b_ref[...], 0.0) + x0_ref[...]
    h_ref[...] = h

    @pl.when(i == 0)
    def _():
        out_ref[...] = jnp.zeros_like(out_ref)

    out_ref[...] += jnp.sum(h, axis=0, keepdims=True)

    @pl.when(i == NB - 1)
    def _():
        out_ref[...] = (
            jnp.dot(out_ref[...] * (1.0 / N), wl_ref[...],
                    preferred_element_type=jnp.float32)
            + bl_ref[...]
        )


_fin_kernel = pl.pallas_call(
    _fin_body,
    grid=(NB,),
    in_specs=[
        pl.BlockSpec((NC, BR, D), lambda i: (0, i, 0)),
        pl.BlockSpec((BR, D), lambda i: (i, 0)),
        pl.BlockSpec((BR, 16), lambda i: (i, 0)),
        pl.BlockSpec((1, D), lambda i: (0, 0)),
        pl.BlockSpec((BR, D), lambda i: (i, 0)),
        pl.BlockSpec((D, D), lambda i: (0, 0)),
        pl.BlockSpec((1, D), lambda i: (0, 0)),
    ],
    out_specs=[
        pl.BlockSpec((BR, D), lambda i: (i, 0)),
        pl.BlockSpec((1, D), lambda i: (0, 0)),
    ],
    out_shape=[
        jax.ShapeDtypeStruct((N, D), jnp.float32),
        jax.ShapeDtypeStruct((1, D), jnp.float32),
    ],
)


def kernel(graph_x, graph_edge, W1, b1, W2, b2, W3, b3, Wl, bl):
    edge = graph_edge.astype(jnp.int32)
    src = edge[0]
    dst = edge[1]

    deg = _deg_from_raw(_deg_kernel()(jnp.ones((N, D), jnp.float32), dst))
    g1, dinv16 = _pre_kernel(graph_x, W1, deg)
    p1 = _scat_kernel()(g1, src, dst)
    g2 = _mid_kernel(p1, g1, dinv16, b1.reshape(1, D), W2)
    p2 = _scat_kernel()(g2, src, dst)
    g3 = _mid_kernel(p2, g2, dinv16, b2.reshape(1, D), W3)
    p3 = _scat_kernel()(g3, src, dst)
    h, out = _fin_kernel(
        p3, g3, dinv16, b3.reshape(1, D), graph_x, Wl, bl.reshape(1, D)
    )
    return (h, out)


# speedup vs baseline: 1.0644x; 1.0076x over previous
import functools

import jax
import jax.numpy as jnp
from jax import lax
from jax.experimental import pallas as pl
from jax.experimental.pallas import tpu as pltpu
from jax.experimental.pallas import tpu_sc as plsc

N = 10000
D = 128
E = 320000

NC = 2
NS = 16
NW = NC * NS
EPW = E // NW
RSTRIDE = 624
RSPAN = 640
C = 128
NCHUNK = EPW // C
TAIL = EPW - NCHUNK * C


def _deg_body(ones_hbm, dst_hbm, out_hbm, di0, di1, di2, di3, dt,
              ones_v, acc, is0, is1, is2, is3, ss0, ss1):
    didx = (di0, di1, di2, di3)
    isem = (is0, is1, is2, is3)
    ssem = (ss0, ss1)
    c = lax.axis_index("c")
    s = lax.axis_index("s")
    base = (c * NS + s) * EPW
    r0 = pl.multiple_of(s * RSTRIDE, 8)
    pltpu.sync_copy(ones_hbm.at[pl.ds(0, C)], ones_v)
    pltpu.sync_copy(ones_hbm.at[pl.ds(r0, RSPAN)], acc.at[pl.ds(r0, RSPAN)])
    plsc.subcore_barrier()

    def ioff(ch):
        return pl.multiple_of(base + ch * C, 8)

    def idx_sync(ch, q):
        pltpu.sync_copy(dst_hbm.at[pl.ds(ioff(ch), C)], didx[q])

    def idx_start(ch, q):
        pltpu.async_copy(dst_hbm.at[pl.ds(ioff(ch), C)], didx[q], isem[q])

    def idx_wait(ch, q):
        pltpu.make_async_copy(dst_hbm.at[pl.ds(ioff(ch), C)], didx[q], isem[q]).wait()

    def scat_start(q, r):
        pltpu.async_copy(ones_v, acc.at[didx[q]], ssem[r], add=True)

    def scat_wait(q, r):
        pltpu.make_async_copy(ones_v, acc.at[didx[q]], ssem[r]).wait()

    idx_sync(0, 0)
    idx_sync(1, 1)
    idx_start(2, 2)
    idx_start(3, 3)
    scat_start(0, 0)
    scat_start(1, 1)

    def body(g, carry):
        for b in range(4):
            ch = 4 * g + 2 + b
            scat_wait(b % 4, b % 2)
            idx_wait(ch, (2 + b) % 4)
            scat_start((2 + b) % 4, b % 2)
            idx_start(ch + 2, b % 4)
        return carry

    lax.fori_loop(0, (NCHUNK - 6) // 4, body, 0)
    for k in range(4):
        ch = NCHUNK - 4 + k
        scat_wait((ch - 2) % 4, ch % 2)
        idx_wait(ch, ch % 4)
        scat_start(ch % 4, ch % 2)
        if k < 2:
            idx_start(ch + 2, (ch + 2) % 4)
    scat_wait((NCHUNK - 2) % 4, (NCHUNK - 2) % 2)
    scat_wait((NCHUNK - 1) % 4, (NCHUNK - 1) % 2)
    offt = pl.multiple_of(base + NCHUNK * C, 8)
    pltpu.sync_copy(dst_hbm.at[pl.ds(offt, TAIL)], dt)
    pltpu.sync_copy(ones_v.at[pl.ds(0, TAIL)], acc.at[dt], add=True)
    plsc.subcore_barrier()
    pltpu.sync_copy(
        acc.at[pl.ds(r0, RSPAN)], out_hbm.at[c, pl.ds(r0, RSPAN)]
    )


@functools.cache
def _deg_kernel():
    mesh = plsc.VectorSubcoreMesh(
        core_axis_name="c", subcore_axis_name="s", num_cores=NC, num_subcores=NS
    )
    return pl.kernel(
        _deg_body,
        out_type=jax.ShapeDtypeStruct((NC, N, D), jnp.float32),
        mesh=mesh,
        scratch_types=[
            pltpu.VMEM((C,), jnp.int32),
            pltpu.VMEM((C,), jnp.int32),
            pltpu.VMEM((C,), jnp.int32),
            pltpu.VMEM((C,), jnp.int32),
            pltpu.VMEM((TAIL,), jnp.int32),
            pltpu.VMEM((C, D), jnp.float32),
            pltpu.VMEM_SHARED((N, D), jnp.float32),
            pltpu.SemaphoreType.DMA,
            pltpu.SemaphoreType.DMA,
            pltpu.SemaphoreType.DMA,
            pltpu.SemaphoreType.DMA,
            pltpu.SemaphoreType.DMA,
            pltpu.SemaphoreType.DMA,
        ],
    )


def _deg_from_raw(raw):
    return (raw[0, :, 0] + raw[1, :, 0] - 2.0).reshape(N, 1)


def _scat_body(
    g_hbm, src_hbm, dst_hbm, out_hbm,
    si0, si1, si2, si3, di0, di1, di2, di3, st, dt,
    rows0, rows1, rowst,
    acc, is0, is1, is2, is3, gs0, gs1, ss0, ss1, ts,
):
    sidx = (si0, si1, si2, si3)
    didx = (di0, di1, di2, di3)
    rows = (rows0, rows1)
    isem = (is0, is1, is2, is3)
    gsem = (gs0, gs1)
    ssem = (ss0, ss1)
    c = lax.axis_index("c")
    s = lax.axis_index("s")
    base = (c * NS + s) * EPW

    def ioff(ch):
        return pl.multiple_of(base + ch * C, 8)

    def idx_sync(ch, q):
        pltpu.sync_copy(src_hbm.at[pl.ds(ioff(ch), C)], sidx[q])
        pltpu.sync_copy(dst_hbm.at[pl.ds(ioff(ch), C)], didx[q])

    def idx_start(ch, q):
        pltpu.async_copy(src_hbm.at[pl.ds(ioff(ch), C)], sidx[q], isem[q])
        pltpu.async_copy(dst_hbm.at[pl.ds(ioff(ch), C)], didx[q], isem[q])

    def idx_wait(ch, q):
        pltpu.make_async_copy(src_hbm.at[pl.ds(ioff(ch), C)], sidx[q], isem[q]).wait()
        pltpu.make_async_copy(dst_hbm.at[pl.ds(ioff(ch), C)], didx[q], isem[q]).wait()

    def gather_start(q, r):
        pltpu.async_copy(g_hbm.at[sidx[q]], rows[r], gsem[r])

    def gather_wait(q, r):
        pltpu.make_async_copy(g_hbm.at[sidx[q]], rows[r], gsem[r]).wait()

    def scat_start(q, r):
        pltpu.async_copy(rows[r], acc.at[didx[q]], ssem[r], add=True)

    def scat_wait(q, r):
        pltpu.make_async_copy(rows[r], acc.at[didx[q]], ssem[r]).wait()

    idx_sync(0, 0)
    idx_sync(1, 1)
    idx_start(2, 2)
    idx_start(3, 3)
    offt = pl.multiple_of(base + NCHUNK * C, 8)
    pltpu.async_copy(src_hbm.at[pl.ds(offt, TAIL)], st, ts)
    pltpu.async_copy(dst_hbm.at[pl.ds(offt, TAIL)], dt, ts)
    gather_start(0, 0)
    r0 = pl.multiple_of(s * RSTRIDE, 8)
    pltpu.sync_copy(g_hbm.at[pl.ds(r0, RSPAN)], acc.at[pl.ds(r0, RSPAN)])
    plsc.subcore_barrier()
    gather_start(1, 1)
    gather_wait(0, 0)
    scat_start(0, 0)

    def body(g, carry):
        for b in range(4):
            ch = 4 * g + 2 + b
            rcur = b % 2
            qcur = (2 + b) % 4
            scat_wait(b % 4, rcur)
            idx_wait(ch, qcur)
            gather_start(qcur, rcur)
            gather_wait((1 + b) % 4, 1 - rcur)
            scat_start((1 + b) % 4, 1 - rcur)
            idx_start(ch + 2, b % 4)
        return carry

    lax.fori_loop(0, (NCHUNK - 6) // 4, body, 0)

    for k in range(4):
        ch = NCHUNK - 4 + k
        rcur = ch % 2
        qcur = ch % 4
        scat_wait((ch - 2) % 4, rcur)
        idx_wait(ch, qcur)
        gather_start(qcur, rcur)
        gather_wait((ch - 1) % 4, 1 - rcur)
        scat_start((ch - 1) % 4, 1 - rcur)
        if k < 2:
            idx_start(ch + 2, (ch + 2) % 4)
    gather_wait((NCHUNK - 1) % 4, (NCHUNK - 1) % 2)
    scat_start((NCHUNK - 1) % 4, (NCHUNK - 1) % 2)
    pltpu.make_async_copy(src_hbm.at[pl.ds(offt, TAIL)], st, ts).wait()
    pltpu.make_async_copy(dst_hbm.at[pl.ds(offt, TAIL)], dt, ts).wait()
    pltpu.async_copy(g_hbm.at[st], rowst, ts)
    scat_wait((NCHUNK - 2) % 4, (NCHUNK - 2) % 2)
    scat_wait((NCHUNK - 1) % 4, (NCHUNK - 1) % 2)
    pltpu.make_async_copy(g_hbm.at[st], rowst, ts).wait()
    pltpu.sync_copy(rowst, acc.at[dt], add=True)
    plsc.subcore_barrier()
    pltpu.sync_copy(
        acc.at[pl.ds(r0, RSPAN)], out_hbm.at[c, pl.ds(r0, RSPAN)]
    )


@functools.cache
def _scat_kernel():
    mesh = plsc.VectorSubcoreMesh(
        core_axis_name="c", subcore_axis_name="s", num_cores=NC, num_subcores=NS
    )
    return pl.kernel(
        _scat_body,
        out_type=jax.ShapeDtypeStruct((NC, N, D), jnp.float32),
        mesh=mesh,
        scratch_types=[
            pltpu.VMEM((C,), jnp.int32),
            pltpu.VMEM((C,), jnp.int32),
            pltpu.VMEM((C,), jnp.int32),
            pltpu.VMEM((C,), jnp.int32),
            pltpu.VMEM((C,), jnp.int32),
            pltpu.VMEM((C,), jnp.int32),
            pltpu.VMEM((C,), jnp.int32),
            pltpu.VMEM((C,), jnp.int32),
            pltpu.VMEM((TAIL,), jnp.int32),
            pltpu.VMEM((TAIL,), jnp.int32),
            pltpu.VMEM((C, D), jnp.float32),
            pltpu.VMEM((C, D), jnp.float32),
            pltpu.VMEM((TAIL, D), jnp.float32),
            pltpu.VMEM_SHARED((N, D), jnp.float32),
            pltpu.SemaphoreType.DMA,
            pltpu.SemaphoreType.DMA,
            pltpu.SemaphoreType.DMA,
            pltpu.SemaphoreType.DMA,
            pltpu.SemaphoreType.DMA,
            pltpu.SemaphoreType.DMA,
            pltpu.SemaphoreType.DMA,
            pltpu.SemaphoreType.DMA,
            pltpu.SemaphoreType.DMA,
        ],
    )


BR = 400
NB = N // BR


def _pre_body(x_ref, w_ref, deg_ref, g_ref, dinv_ref):
    dinv = lax.rsqrt(deg_ref[...] + 1.0)
    g_ref[...] = (
        jnp.dot(x_ref[...], w_ref[...], preferred_element_type=jnp.float32) * dinv
    )
    dinv_ref[...] = jnp.broadcast_to(dinv, (BR, 16))


_pre_kernel = pl.pallas_call(
    _pre_body,
    grid=(NB,),
    in_specs=[
        pl.BlockSpec((BR, D), lambda i: (i, 0)),
        pl.BlockSpec((D, D), lambda i: (0, 0)),
        pl.BlockSpec((BR, 1), lambda i: (i, 0)),
    ],
    out_specs=[
        pl.BlockSpec((BR, D), lambda i: (i, 0)),
        pl.BlockSpec((BR, 16), lambda i: (i, 0)),
    ],
    out_shape=[
        jax.ShapeDtypeStruct((N, D), jnp.float32),
        jax.ShapeDtypeStruct((N, 16), jnp.float32),
    ],
)


def _mid_body(part_ref, g_ref, dinv_ref, b_ref, w_ref, gn_ref):
    dinv = dinv_ref[:, 0:1]
    agg = part_ref[0] + part_ref[1] - g_ref[...]
    xn = jnp.maximum(agg * dinv + b_ref[...], 0.0)
    gn_ref[...] = (
        jnp.dot(xn, w_ref[...], preferred_element_type=jnp.float32) * dinv
    )


_mid_kernel = pl.pallas_call(
    _mid_body,
    grid=(NB,),
    in_specs=[
        pl.BlockSpec((NC, BR, D), lambda i: (0, i, 0)),
        pl.BlockSpec((BR, D), lambda i: (i, 0)),
        pl.BlockSpec((BR, 16), lambda i: (i, 0)),
        pl.BlockSpec((1, D), lambda i: (0, 0)),
        pl.BlockSpec((D, D), lambda i: (0, 0)),
    ],
    out_specs=pl.BlockSpec((BR, D), lambda i: (i, 0)),
    out_shape=jax.ShapeDtypeStruct((N, D), jnp.float32),
)


def _fin_body(part_ref, g_ref, dinv_ref, b_ref, x0_ref, wl_ref, bl_ref,
              h_ref, out_ref):
    i = pl.program_id(0)
    dinv = dinv_ref[:, 0:1]
    agg = part_ref[0] + part_ref[1] - g_ref[...]
    h = jnp.maximum(agg * dinv + b_ref[...], 0.0) + x0_ref[...]
    h_ref[...] = h

    @pl.when(i == 0)
    def _():
        out_ref[...] = jnp.zeros_like(out_ref)

    out_ref[...] += jnp.sum(h, axis=0, keepdims=True)

    @pl.when(i == NB - 1)
    def _():
        out_ref[...] = (
            jnp.dot(out_ref[...] * (1.0 / N), wl_ref[...],
                    preferred_element_type=jnp.float32)
            + bl_ref[...]
        )


_fin_kernel = pl.pallas_call(
    _fin_body,
    grid=(NB,),
    in_specs=[
        pl.BlockSpec((NC, BR, D), lambda i: (0, i, 0)),
        pl.BlockSpec((BR, D), lambda i: (i, 0)),
        pl.BlockSpec((BR, 16), lambda i: (i, 0)),
        pl.BlockSpec((1, D), lambda i: (0, 0)),
        pl.BlockSpec((BR, D), lambda i: (i, 0)),
        pl.BlockSpec((D, D), lambda i: (0, 0)),
        pl.BlockSpec((1, D), lambda i: (0, 0)),
    ],
    out_specs=[
        pl.BlockSpec((BR, D), lambda i: (i, 0)),
        pl.BlockSpec((1, D), lambda i: (0, 0)),
    ],
    out_shape=[
        jax.ShapeDtypeStruct((N, D), jnp.float32),
        jax.ShapeDtypeStruct((1, D), jnp.float32),
    ],
)


def kernel(graph_x, graph_edge, W1, b1, W2, b2, W3, b3, Wl, bl):
    edge = graph_edge.astype(jnp.int32)
    src = edge[0]
    dst = edge[1]

    deg = _deg_from_raw(_deg_kernel()(jnp.ones((N, D), jnp.float32), dst))
    g1, dinv16 = _pre_kernel(graph_x, W1, deg)
    p1 = _scat_kernel()(g1, src, dst)
    g2 = _mid_kernel(p1, g1, dinv16, b1.reshape(1, D), W2)
    p2 = _scat_kernel()(g2, src, dst)
    g3 = _mid_kernel(p2, g2, dinv16, b2.reshape(1, D), W3)
    p3 = _scat_kernel()(g3, src, dst)
    h, out = _fin_kernel(
        p3, g3, dinv16, b3.reshape(1, D), graph_x, Wl, bl.reshape(1, D)
    )
    return (h, out)
